# Initial kernel scaffold; baseline (speedup 1.0000x reference)
import jax
import jax.numpy as jnp
from jax.experimental import pallas as pl


def kernel(h, edge_index, edge_attr, W1, b1, W2, b2, eps):
    raise NotImplementedError("write your pallas kernel here")


# speedup vs baseline: 4.7218x; 4.7218x over previous
import functools

import jax
import jax.numpy as jnp
from jax import lax
from jax.experimental import pallas as pl
from jax.experimental.pallas import tpu as pltpu
from jax.experimental.pallas import tpu_sc as plsc

N = 10000
N_PAD = 10240
E = 320000
D = 128
CHUNK = 128
ROWS = E // CHUNK
NW = 32


def _sc_aggregate(h, src1d, dst1d, attr3d, zeros):
    mesh = plsc.VectorSubcoreMesh(core_axis_name="c", subcore_axis_name="s")

    @functools.partial(
        pl.kernel,
        mesh=mesh,
        out_type=jax.ShapeDtypeStruct((2, N_PAD, D), jnp.float32),
        scratch_types=[
            pltpu.VMEM((CHUNK,), jnp.int32),
            pltpu.VMEM((CHUNK,), jnp.int32),
            pltpu.VMEM((CHUNK, D), jnp.float32),
            pltpu.VMEM((CHUNK, D), jnp.float32),
            pltpu.VMEM_SHARED((N_PAD, D), jnp.float32),
            pltpu.SemaphoreType.DMA,
        ],
    )
    def k(h_hbm, src_hbm, dst_hbm, attr_hbm, z_hbm, out_hbm,
          src_v, dst_v, hrows_v, attr_v, aggr_sh, sem):
        c = lax.axis_index("c")
        s = lax.axis_index("s")
        w = c * 16 + s

        rows_per_tile = N_PAD // 16
        pltpu.sync_copy(z_hbm.at[pl.ds(s * rows_per_tile, rows_per_tile)],
                        aggr_sh.at[pl.ds(s * rows_per_tile, rows_per_tile)])
        plsc.subcore_barrier()

        start = (w * ROWS) // NW
        end = ((w + 1) * ROWS) // NW

        def body(r, carry):
            pltpu.sync_copy(src_hbm.at[pl.ds(r * CHUNK, CHUNK)], src_v)
            pltpu.sync_copy(dst_hbm.at[pl.ds(r * CHUNK, CHUNK)], dst_v)
            gath = pltpu.async_copy(h_hbm.at[src_v], hrows_v, sem)
            pltpu.sync_copy(attr_hbm.at[r], attr_v)
            gath.wait()
            pltpu.sync_copy(attr_v, aggr_sh.at[dst_v], add=True)
            pltpu.sync_copy(hrows_v, aggr_sh.at[dst_v], add=True)
            return carry

        lax.fori_loop(start, end, body, 0)
        plsc.subcore_barrier()

        pltpu.sync_copy(aggr_sh.at[pl.ds(s * rows_per_tile, rows_per_tile)],
                        out_hbm.at[c, pl.ds(s * rows_per_tile, rows_per_tile)])

    return k(h, src1d, dst1d, attr3d, zeros)


def _tc_mlp_body(h_ref, p_ref, w1_ref, b1_ref, w2_ref, b2_ref, eps_ref,
                 out_ref):
    aggr = p_ref[0] + p_ref[1]
    hid = jnp.dot(aggr, w1_ref[...], preferred_element_type=jnp.float32)
    hid = jnp.maximum(hid + b1_ref[...], 0.0)
    out = jnp.dot(hid, w2_ref[...], preferred_element_type=jnp.float32)
    out_ref[...] = (1.0 + eps_ref[0]) * h_ref[...] + out + b2_ref[...]


def _tc_mlp(h, partials, W1, b1, W2, b2, eps):
    BR = 256
    grid = (pl.cdiv(N, BR),)
    return pl.pallas_call(
        _tc_mlp_body,
        grid=grid,
        in_specs=[
            pl.BlockSpec((BR, D), lambda i: (i, 0)),
            pl.BlockSpec((2, BR, D), lambda i: (0, i, 0)),
            pl.BlockSpec((D, 2 * D), lambda i: (0, 0)),
            pl.BlockSpec((1, 2 * D), lambda i: (0, 0)),
            pl.BlockSpec((2 * D, D), lambda i: (0, 0)),
            pl.BlockSpec((1, D), lambda i: (0, 0)),
            pl.BlockSpec(memory_space=pltpu.SMEM),
        ],
        out_specs=pl.BlockSpec((BR, D), lambda i: (i, 0)),
        out_shape=jax.ShapeDtypeStruct((N, D), jnp.float32),
    )(h, partials, W1, b1.reshape(1, 2 * D), W2, b2.reshape(1, D), eps)


def kernel(h, edge_index, edge_attr, W1, b1, W2, b2, eps):
    src1d = edge_index[0]
    dst1d = edge_index[1]
    attr3d = edge_attr.reshape(ROWS, CHUNK, D)
    zeros = jnp.zeros((N_PAD, D), jnp.float32)
    partials = _sc_aggregate(h, src1d, dst1d, attr3d, zeros)
    return _tc_mlp(h, partials, W1, b1, W2, b2, eps)


# speedup vs baseline: 6.4621x; 1.3686x over previous
import functools

import jax
import jax.numpy as jnp
from jax import lax
from jax.experimental import pallas as pl
from jax.experimental.pallas import tpu as pltpu
from jax.experimental.pallas import tpu_sc as plsc

N = 10000
N_ACC = 10112
E = 320000
D = 128
CHUNK = 128
ROWS = E // CHUNK
NW = 32
KPW = 80
ROWS_PAD = NW * KPW


def _sc_aggregate(h, src1d, dst1d, attr3d, zeros):
    mesh = plsc.VectorSubcoreMesh(core_axis_name="c", subcore_axis_name="s")

    @functools.partial(
        pl.kernel,
        mesh=mesh,
        out_type=jax.ShapeDtypeStruct((2, N_ACC, D), jnp.float32),
        scratch_types=[
            pltpu.VMEM((2, CHUNK), jnp.int32),
            pltpu.VMEM((2, CHUNK), jnp.int32),
            pltpu.VMEM((2, CHUNK, D), jnp.float32),
            pltpu.VMEM((CHUNK, D), jnp.float32),
            pltpu.VMEM_SHARED((N_ACC, D), jnp.float32),
            pltpu.SemaphoreType.DMA,
            pltpu.SemaphoreType.DMA,
            pltpu.SemaphoreType.DMA,
        ],
    )
    def k(h_hbm, src_hbm, dst_hbm, attr_hbm, z_hbm, out_hbm,
          src_v, dst_v, hrows_v, attr_v, aggr_sh, sem_g, sem_a, sem_i):
        c = lax.axis_index("c")
        s = lax.axis_index("s")
        w = c * 16 + s

        rows_per_tile = N_ACC // 16
        pltpu.sync_copy(z_hbm.at[pl.ds(s * rows_per_tile, rows_per_tile)],
                        aggr_sh.at[pl.ds(s * rows_per_tile, rows_per_tile)])
        plsc.subcore_barrier()

        start = w * KPW
        end = jnp.minimum(start + KPW, ROWS)

        def issue_idx(r):
            p = r % 2
            sl = pl.ds(r * CHUNK, CHUNK)
            pltpu.async_copy(src_hbm.at[sl], src_v.at[p], sem_i)
            pltpu.async_copy(dst_hbm.at[sl], dst_v.at[p], sem_i)

        def wait_idx(r):
            p = r % 2
            sl = pl.ds(r * CHUNK, CHUNK)
            pltpu.make_async_copy(src_hbm.at[sl], src_v.at[p], sem_i).wait()
            pltpu.make_async_copy(dst_hbm.at[sl], dst_v.at[p], sem_i).wait()

        def issue_gather(r):
            p = r % 2
            pltpu.async_copy(h_hbm.at[src_v.at[p]], hrows_v.at[p], sem_g)

        def wait_gather(r):
            p = r % 2
            pltpu.make_async_copy(h_hbm.at[src_v.at[p]], hrows_v.at[p],
                                  sem_g).wait()

        @pl.when(start < end)
        def _():
            p = start % 2
            sl = pl.ds(start * CHUNK, CHUNK)
            pltpu.sync_copy(src_hbm.at[sl], src_v.at[p])
            pltpu.sync_copy(dst_hbm.at[sl], dst_v.at[p])
            pltpu.async_copy(attr_hbm.at[start], attr_v, sem_a)
            issue_gather(start)

        def body(r, carry):
            p = r % 2

            @pl.when(r + 1 < end)
            def _():
                issue_idx(r + 1)

            pltpu.make_async_copy(attr_hbm.at[r], attr_v, sem_a).wait()
            pltpu.sync_copy(attr_v, aggr_sh.at[dst_v.at[p]], add=True)

            @pl.when(r + 1 < end)
            def _():
                pltpu.async_copy(attr_hbm.at[r + 1], attr_v, sem_a)

            wait_gather(r)

            @pl.when(r + 1 < end)
            def _():
                wait_idx(r + 1)
                issue_gather(r + 1)

            pltpu.sync_copy(hrows_v.at[p], aggr_sh.at[dst_v.at[p]], add=True)
            return carry

        lax.fori_loop(start, end, body, 0)
        plsc.subcore_barrier()

        pltpu.sync_copy(aggr_sh.at[pl.ds(s * rows_per_tile, rows_per_tile)],
                        out_hbm.at[c, pl.ds(s * rows_per_tile, rows_per_tile)])

    return k(h, src1d, dst1d, attr3d, zeros)


def _tc_mlp_body(h_ref, p_ref, w1_ref, b1_ref, w2_ref, b2_ref, eps_ref,
                 out_ref):
    aggr = p_ref[0] + p_ref[1]
    hid = jnp.dot(aggr, w1_ref[...], preferred_element_type=jnp.float32)
    hid = jnp.maximum(hid + b1_ref[...], 0.0)
    out = jnp.dot(hid, w2_ref[...], preferred_element_type=jnp.float32)
    out_ref[...] = (1.0 + eps_ref[0]) * h_ref[...] + out + b2_ref[...]


def _tc_mlp(h, partials, W1, b1, W2, b2, eps):
    BR = 256
    grid = (pl.cdiv(N, BR),)
    return pl.pallas_call(
        _tc_mlp_body,
        grid=grid,
        in_specs=[
            pl.BlockSpec((BR, D), lambda i: (i, 0)),
            pl.BlockSpec((2, BR, D), lambda i: (0, i, 0)),
            pl.BlockSpec((D, 2 * D), lambda i: (0, 0)),
            pl.BlockSpec((1, 2 * D), lambda i: (0, 0)),
            pl.BlockSpec((2 * D, D), lambda i: (0, 0)),
            pl.BlockSpec((1, D), lambda i: (0, 0)),
            pl.BlockSpec(memory_space=pltpu.SMEM),
        ],
        out_specs=pl.BlockSpec((BR, D), lambda i: (i, 0)),
        out_shape=jax.ShapeDtypeStruct((N, D), jnp.float32),
    )(h, partials, W1, b1.reshape(1, 2 * D), W2, b2.reshape(1, D), eps)


def kernel(h, edge_index, edge_attr, W1, b1, W2, b2, eps):
    pad = jnp.zeros((ROWS_PAD * CHUNK - E,), jnp.int32)
    src1d = jnp.concatenate([edge_index[0], pad])
    dst1d = jnp.concatenate([edge_index[1], pad])
    attr3d = edge_attr.reshape(ROWS, CHUNK, D)
    zeros = jnp.zeros((N_ACC, D), jnp.float32)
    partials = _sc_aggregate(h, src1d, dst1d, attr3d, zeros)
    return _tc_mlp(h, partials, W1, b1, W2, b2, eps)


# speedup vs baseline: 7.9642x; 1.2324x over previous
import functools

import jax
import jax.numpy as jnp
from jax import lax
from jax.experimental import pallas as pl
from jax.experimental.pallas import tpu as pltpu
from jax.experimental.pallas import tpu_sc as plsc

N = 10000
N_ACC = 10112
E = 320000
D = 128
CHUNK = 80
ROWS = E // CHUNK
NW = 32
KPW = ROWS // NW


def _sc_aggregate(h, src1d, dst1d, attr2d, zeros):
    mesh = plsc.VectorSubcoreMesh(core_axis_name="c", subcore_axis_name="s")

    @functools.partial(
        pl.kernel,
        mesh=mesh,
        out_type=jax.ShapeDtypeStruct((2, N_ACC, D), jnp.float32),
        scratch_types=[
            pltpu.VMEM((2, CHUNK), jnp.int32),
            pltpu.VMEM((2, CHUNK), jnp.int32),
            pltpu.VMEM((2, CHUNK, D), jnp.float32),
            pltpu.VMEM((2, CHUNK, D), jnp.float32),
            pltpu.VMEM_SHARED((N_ACC, D), jnp.float32),
            pltpu.SemaphoreType.DMA,
            pltpu.SemaphoreType.DMA,
            pltpu.SemaphoreType.DMA,
        ],
    )
    def k(h_hbm, src_hbm, dst_hbm, attr_hbm, z_hbm, out_hbm,
          src_v, dst_v, hrows_v, attr_v, aggr_sh, sem_g, sem_a, sem_i):
        c = lax.axis_index("c")
        s = lax.axis_index("s")
        w = c * 16 + s

        rows_per_tile = N_ACC // 16
        pltpu.sync_copy(z_hbm.at[pl.ds(s * rows_per_tile, rows_per_tile)],
                        aggr_sh.at[pl.ds(s * rows_per_tile, rows_per_tile)])
        plsc.subcore_barrier()

        start = w * KPW
        end = start + KPW

        def issue_idx(r):
            p = r % 2
            sl = pl.ds(r * CHUNK, CHUNK)
            pltpu.async_copy(src_hbm.at[sl], src_v.at[p], sem_i)
            pltpu.async_copy(dst_hbm.at[sl], dst_v.at[p], sem_i)

        def wait_idx(r):
            p = r % 2
            sl = pl.ds(r * CHUNK, CHUNK)
            pltpu.make_async_copy(src_hbm.at[sl], src_v.at[p], sem_i).wait()
            pltpu.make_async_copy(dst_hbm.at[sl], dst_v.at[p], sem_i).wait()

        def issue_gather(r):
            p = r % 2
            pltpu.async_copy(h_hbm.at[src_v.at[p]], hrows_v.at[p], sem_g)

        def wait_gather(r):
            p = r % 2
            pltpu.make_async_copy(h_hbm.at[src_v.at[p]], hrows_v.at[p],
                                  sem_g).wait()

        def issue_attr(r):
            p = r % 2
            sl = pl.ds(r * CHUNK, CHUNK)
            pltpu.async_copy(attr_hbm.at[sl], attr_v.at[p], sem_a)

        def wait_attr(r):
            p = r % 2
            sl = pl.ds(r * CHUNK, CHUNK)
            pltpu.make_async_copy(attr_hbm.at[sl], attr_v.at[p],
                                  sem_a).wait()

        p0 = start % 2
        sl0 = pl.ds(start * CHUNK, CHUNK)
        pltpu.sync_copy(src_hbm.at[sl0], src_v.at[p0])
        pltpu.sync_copy(dst_hbm.at[sl0], dst_v.at[p0])
        issue_attr(start)
        issue_gather(start)

        def body(r, carry):
            p = r % 2

            @pl.when(r + 1 < end)
            def _():
                issue_idx(r + 1)
                issue_attr(r + 1)

            wait_attr(r)
            pltpu.sync_copy(attr_v.at[p], aggr_sh.at[dst_v.at[p]], add=True)
            wait_gather(r)

            @pl.when(r + 1 < end)
            def _():
                wait_idx(r + 1)
                issue_gather(r + 1)

            pltpu.sync_copy(hrows_v.at[p], aggr_sh.at[dst_v.at[p]], add=True)
            return carry

        lax.fori_loop(start, end, body, 0)
        plsc.subcore_barrier()

        pltpu.sync_copy(aggr_sh.at[pl.ds(s * rows_per_tile, rows_per_tile)],
                        out_hbm.at[c, pl.ds(s * rows_per_tile, rows_per_tile)])

    return k(h, src1d, dst1d, attr2d, zeros)


def _tc_mlp_body(h_ref, p_ref, w1_ref, b1_ref, w2_ref, b2_ref, eps_ref,
                 out_ref):
    aggr = p_ref[0] + p_ref[1]
    hid = jnp.dot(aggr, w1_ref[...], preferred_element_type=jnp.float32)
    hid = jnp.maximum(hid + b1_ref[...], 0.0)
    out = jnp.dot(hid, w2_ref[...], preferred_element_type=jnp.float32)
    out_ref[...] = (1.0 + eps_ref[0]) * h_ref[...] + out + b2_ref[...]


def _tc_mlp(h, partials, W1, b1, W2, b2, eps):
    BR = 256
    grid = (pl.cdiv(N, BR),)
    return pl.pallas_call(
        _tc_mlp_body,
        grid=grid,
        in_specs=[
            pl.BlockSpec((BR, D), lambda i: (i, 0)),
            pl.BlockSpec((2, BR, D), lambda i: (0, i, 0)),
            pl.BlockSpec((D, 2 * D), lambda i: (0, 0)),
            pl.BlockSpec((1, 2 * D), lambda i: (0, 0)),
            pl.BlockSpec((2 * D, D), lambda i: (0, 0)),
            pl.BlockSpec((1, D), lambda i: (0, 0)),
            pl.BlockSpec(memory_space=pltpu.SMEM),
        ],
        out_specs=pl.BlockSpec((BR, D), lambda i: (i, 0)),
        out_shape=jax.ShapeDtypeStruct((N, D), jnp.float32),
    )(h, partials, W1, b1.reshape(1, 2 * D), W2, b2.reshape(1, D), eps)


def kernel(h, edge_index, edge_attr, W1, b1, W2, b2, eps):
    src1d = edge_index[0]
    dst1d = edge_index[1]
    zeros = jnp.zeros((N_ACC, D), jnp.float32)
    partials = _sc_aggregate(h, src1d, dst1d, edge_attr, zeros)
    return _tc_mlp(h, partials, W1, b1, W2, b2, eps)
